# Initial kernel scaffold; baseline (speedup 1.0000x reference)
import functools

import jax
import jax.numpy as jnp
from jax import lax
from jax.experimental import pallas as pl
from jax.experimental.pallas import tpu as pltpu
from jax.experimental.pallas import tpu_sc as plsc

B = 16
H = 512
W = 512
NSEG = 1000
BINS = NSEG * NSEG
NC = 2
NS = 16
L = 16
RANGE = 62512
OUT_PAD = NS * RANGE
CH = 64
N_CHUNK = H // CH
VPR = W // L


def _body(seg_hbm, out_hbm, chunk_v, hist_v):
    c = lax.axis_index("c")
    s = lax.axis_index("s")
    lo = s * RANGE
    ones = jnp.ones((L,), jnp.float32)
    zeros = jnp.zeros((L,), jnp.float32)
    lane = lax.iota(jnp.int32, L)

    def batch_body(i, carry):
        b = NC * i + c

        def zero_body(k, carry):
            hist_v[pl.ds(k * L, L)] = zeros
            return carry

        lax.fori_loop(0, RANGE // L, zero_body, 0)

        def chunk_body(ci, carry):
            pltpu.sync_copy(seg_hbm.at[b, pl.ds(ci * CH, CH), :], chunk_v)

            def inner(t, carry):
                r = t >> 5
                j = t & (VPR - 1)
                is_last = j == VPR - 1
                base = j * L - jnp.where(is_last, 1, 0)
                a = chunk_v[r, pl.ds(base, L)]
                d = chunk_v[r, pl.ds(base + 1, L)]
                valid = (a != d) & ~(is_last & (lane == 0))
                loc1 = a * NSEG + d - lo
                m1 = valid & (loc1 >= 0) & (loc1 < RANGE)
                plsc.addupdate_scatter(hist_v, [loc1], ones, mask=m1)
                loc2 = d * NSEG + a - lo
                m2 = valid & (loc2 >= 0) & (loc2 < RANGE)
                plsc.addupdate_scatter(hist_v, [loc2], ones, mask=m2)
                return carry

            lax.fori_loop(0, CH * VPR, inner, 0)
            return carry

        lax.fori_loop(0, N_CHUNK, chunk_body, 0)
        pltpu.sync_copy(hist_v, out_hbm.at[b, pl.ds(lo, RANGE)])
        return carry

    lax.fori_loop(0, B // NC, batch_body, 0)


def _sc_histogram(segments, interpret=False):
    mesh = plsc.VectorSubcoreMesh(
        core_axis_name="c", subcore_axis_name="s", num_cores=NC, num_subcores=NS
    )
    return pl.kernel(
        _body,
        out_type=jax.ShapeDtypeStruct((B, OUT_PAD), jnp.float32),
        mesh=mesh,
        scratch_types=[
            pltpu.VMEM((CH, W), jnp.int32),
            pltpu.VMEM((RANGE,), jnp.float32),
        ],
        interpret=interpret,
    )(segments)


@jax.jit
def kernel(segments):
    out_flat = _sc_histogram(segments)
    return out_flat[:, :BINS].reshape(B, NSEG, NSEG)


# speedup vs baseline: 2.0800x; 2.0800x over previous
import functools

import numpy as np
import jax
import jax.numpy as jnp
from jax import lax
from jax.experimental import pallas as pl
from jax.experimental.pallas import tpu as pltpu
from jax.experimental.pallas import tpu_sc as plsc

B = 16
H = 512
W = 512
NSEG = 1000
BINS = NSEG * NSEG
NC = 2
NS = 16
NW = NC * NS
L = 16
RANGE = 31264
SLABPAD = NW * RANGE
CH = 32
N_CHUNK = H // CH
VPR = W // L
NV = CH * VPR

_GDN = lax.GatherDimensionNumbers(
    offset_dims=(), collapsed_slice_dims=(0,), start_index_map=(0,)
)


def _rot1(v, perm2d):
    return lax.gather(
        v, perm2d, _GDN, (1,), mode=lax.GatherScatterMode.PROMISE_IN_BOUNDS
    )


_INV1000 = np.float32(0.001000000047497451)


def _div1000(x):
    return (x.astype(jnp.float32) * _INV1000).astype(jnp.int32)


def _round_bf16(x_i32):
    u = plsc.bitcast(x_i32.astype(jnp.float32), jnp.int32)
    t = u + 0x7FFF + ((u >> 16) & 1)
    t = t & jnp.int32(-65536)
    return plsc.bitcast(t, jnp.float32).astype(jnp.int32)


def _sc_body(seg_hbm, out_hbm, chunk_v, hist_v):
    c = lax.axis_index("c")
    s = lax.axis_index("s")
    wid = s * NC + c
    lo = wid * RANGE
    ones = jnp.ones((L,), jnp.float32)
    zeros = jnp.zeros((L,), jnp.float32)
    lane = lax.iota(jnp.int32, L)
    perm2d = ((lane + 1) & (L - 1))[:, None]
    lane15 = lane == L - 1
    million = jnp.int32(1_000_000)

    def zero_slot(slot):
        def zbody(k, carry):
            hist_v[pl.ds(slot * RANGE + k * L, L)] = zeros
            return carry

        lax.fori_loop(0, RANGE // L, zbody, 0)

    for slot in range(3):
        zero_slot(slot)

    def batch_body(b, carry):
        off_b = NSEG * b
        slot_prev = (b + 2) % 3
        slot_cur = b % 3
        slot_next = (b + 1) % 3
        base_prev = slot_prev * RANGE
        base_cur = slot_cur * RANGE
        base_next = slot_next * RANGE

        def chunk_body(ci, carry):
            pltpu.sync_copy(seg_hbm.at[b, pl.ds(ci * CH, CH), :], chunk_v)
            x0 = _round_bf16(chunk_v[0, pl.ds(0, L)] + off_b)

            def inner(t, x1):
                tn = jnp.minimum(t + 1, NV - 1)
                rn = tn >> 5
                jn = tn & (VPR - 1)
                raw_n = chunk_v[rn, pl.ds(pl.multiple_of(jn * L, L), L)]
                x_next = _round_bf16(raw_n + off_b)
                x2 = jnp.where(lane15, _rot1(x_next, perm2d), _rot1(x1, perm2d))
                eb = _div1000(x1)
                flat = x1 * NSEG + x2 - eb * NSEG
                rem0 = flat - eb * million
                neg = rem0 < 0
                big = rem0 >= million
                rem = rem0 + jnp.where(neg, million, 0) - jnp.where(big, million, 0)
                delta = (eb - (b - 1)) - neg.astype(jnp.int32) + big.astype(jnp.int32)
                base = jnp.where(
                    delta == 0, base_prev, jnp.where(delta == 1, base_cur, base_next)
                )
                valid = (x1 != x2) & ~(lane15 & ((t & (VPR - 1)) == VPR - 1))
                u = rem - lo
                m = valid & (u >= 0) & (u < RANGE)
                plsc.addupdate_scatter(hist_v, [base + u], ones, mask=m)
                return x_next

            lax.fori_loop(0, NV, inner, x0)
            return carry

        lax.fori_loop(0, N_CHUNK, chunk_body, 0)

        @pl.when(b >= 1)
        def _flush():
            off = pl.multiple_of((b - 1) * SLABPAD + lo, 8)
            pltpu.sync_copy(
                hist_v.at[pl.ds(base_prev, RANGE)], out_hbm.at[pl.ds(off, RANGE)]
            )
            zero_slot(slot_prev)

        return carry

    lax.fori_loop(0, B, batch_body, 0, unroll=3)
    off = pl.multiple_of((B - 1) * SLABPAD + lo, 8)
    pltpu.sync_copy(hist_v.at[pl.ds(0, RANGE)], out_hbm.at[pl.ds(off, RANGE)])


def _sc_histogram(segments):
    mesh = plsc.VectorSubcoreMesh(
        core_axis_name="c", subcore_axis_name="s", num_cores=NC, num_subcores=NS
    )
    return pl.kernel(
        _sc_body,
        out_type=jax.ShapeDtypeStruct((B * SLABPAD,), jnp.float32),
        mesh=mesh,
        scratch_types=[
            pltpu.VMEM((CH, W), jnp.int32),
            pltpu.VMEM((3 * RANGE,), jnp.float32),
        ],
        compiler_params=pltpu.CompilerParams(needs_layout_passes=False),
    )(segments)


def _sym_body(x_ref, o_ref):
    x = x_ref[0]
    o_ref[0] = x + x.T


def _symmetrize(cnt):
    return pl.pallas_call(
        _sym_body,
        grid=(B,),
        in_specs=[pl.BlockSpec((1, NSEG, NSEG), lambda b: (b, 0, 0))],
        out_specs=pl.BlockSpec((1, NSEG, NSEG), lambda b: (b, 0, 0)),
        out_shape=jax.ShapeDtypeStruct((B, NSEG, NSEG), jnp.float32),
    )(cnt)


@jax.jit
def kernel(segments):
    out_flat = _sc_histogram(segments)
    cnt = out_flat.reshape(B, SLABPAD)[:, :BINS].reshape(B, NSEG, NSEG)
    return _symmetrize(cnt)
